# Initial kernel scaffold; baseline (speedup 1.0000x reference)
import jax
import jax.numpy as jnp
from jax.experimental import pallas as pl


def kernel(x, edge_index, Wl1, Wr1, b1, Wl2, Wr2, b2, Wl3, Wr3, b3):
    raise NotImplementedError("write your pallas kernel here")


# speedup vs baseline: 5.2168x; 5.2168x over previous
import functools
import jax
import jax.numpy as jnp
from jax import lax
from jax.experimental import pallas as pl
from jax.experimental.pallas import tpu as pltpu
from jax.experimental.pallas import tpu_sc as plsc

NN = 10000
NP = 10240
EE = 320000
DD = 128

NC = 2
NS = 16
NW = NC * NS
EPT = EE // NW
CH = 80
NCHUNK = EPT // CH
RPT = NP // NS
W16 = 16
NZB = RPT // CH


def _seg_body(nfeat, with_deg, *refs):
    if with_deg:
        (y_hbm, src_hbm, dst_hbm, out_hbm, deg_hbm,
         sidx, didx, rows, hist1, hist, acc, sem) = refs
    else:
        (y_hbm, src_hbm, dst_hbm, out_hbm,
         sidx, didx, rows, acc, sem) = refs

    cid = lax.axis_index("c")
    sid = lax.axis_index("s")
    wid = sid * NC + cid
    r0 = sid * RPT

    def zrow(r, carry):
        for j in range(nfeat // 16):
            rows[r, pl.ds(j * 16, 16)] = jnp.zeros((16,), jnp.float32)
        return carry
    lax.fori_loop(0, CH, zrow, 0)
    for j in range(NZB):
        pltpu.sync_copy(rows, acc.at[pl.ds(r0 + j * CH, CH)])
    if with_deg:
        def zh(r, carry):
            for j in range(8):
                hist1[pl.ds(r * 128 + j * 16, 16)] = jnp.zeros(
                    (16,), jnp.int32)
            return carry
        lax.fori_loop(0, NP // 128, zh, 0)
    plsc.subcore_barrier()

    base = wid * EPT

    def chunk(i, carry):
        off = pl.multiple_of(base + i * CH, 8)
        pltpu.sync_copy(src_hbm.at[pl.ds(off, CH)], sidx)
        pltpu.sync_copy(dst_hbm.at[pl.ds(off, CH)], didx)
        pltpu.async_copy(y_hbm.at[sidx], rows, sem).wait()
        pltpu.sync_copy(rows, acc.at[didx], add=True)
        if with_deg:
            def bump(k, carry2):
                dvec = didx[pl.ds(k * 16, 16)]
                cnt, lmask = plsc.scan_count(dvec)
                plsc.addupdate_scatter(hist1, [dvec], cnt, mask=lmask)
                return carry2
            lax.fori_loop(0, CH // 16, bump, 0)
        return carry

    lax.fori_loop(0, NCHUNK, chunk, 0)

    plsc.subcore_barrier()
    for j in range(NZB):
        pltpu.sync_copy(acc.at[pl.ds(r0 + j * CH, CH)], rows)
        pltpu.sync_copy(rows, out_hbm.at[cid, pl.ds(r0 + j * CH, CH)])
    if with_deg:
        def h12(r, carry):
            for j in range(8):
                hist[r, pl.ds(j * 16, 16)] = hist1[
                    pl.ds(r * 128 + j * 16, 16)]
            return carry
        lax.fori_loop(0, NP // 128, h12, 0)
        pltpu.sync_copy(hist, deg_hbm.at[cid, sid])


def _make_seg(nfeat, with_deg):
    mesh = plsc.VectorSubcoreMesh(core_axis_name="c", subcore_axis_name="s")
    out_type = [jax.ShapeDtypeStruct((NC, NP, nfeat), jnp.float32)]
    scratch = [
        pltpu.VMEM((CH,), jnp.int32),
        pltpu.VMEM((CH,), jnp.int32),
        pltpu.VMEM((CH, nfeat), jnp.float32),
    ]
    if with_deg:
        out_type.append(jax.ShapeDtypeStruct((NC, NS, NP // 128, 128),
                                             jnp.int32))
        scratch.append(pltpu.VMEM((NP,), jnp.int32))
        scratch.append(pltpu.VMEM((NP // 128, 128), jnp.int32))
    scratch.append(pltpu.VMEM_SHARED((NP, nfeat), jnp.float32))
    scratch.append(pltpu.SemaphoreType.DMA)

    return pl.kernel(
        functools.partial(_seg_body, nfeat, with_deg),
        out_type=tuple(out_type),
        mesh=mesh,
        scratch_types=tuple(scratch),
        compiler_params=pltpu.CompilerParams(needs_layout_passes=False),
    )


BR = 1024
NB = NP // BR


def _mm2_body(x_ref, wl_ref, wr_ref, y_ref, r_ref):
    xb = x_ref[...]
    y_ref[...] = jnp.dot(xb, wl_ref[...], preferred_element_type=jnp.float32)
    r_ref[...] = jnp.dot(xb, wr_ref[...], preferred_element_type=jnp.float32)


def _deg_inv(dp_ref):
    deg = jnp.sum(dp_ref[...], axis=1, keepdims=True).astype(jnp.float32)
    return 1.0 / jnp.maximum(deg, 1.0)


def _combine_mm2_body(p_ref, dp_ref, r_ref, b_ref, wl_ref, wr_ref,
                      y_ref, rr_ref):
    inv = _deg_inv(dp_ref)
    h = jnp.maximum((p_ref[0] + p_ref[1]) * inv + r_ref[...] + b_ref[...],
                    0.0)
    y_ref[...] = jnp.dot(h, wl_ref[...], preferred_element_type=jnp.float32)
    rr_ref[...] = jnp.dot(h, wr_ref[...], preferred_element_type=jnp.float32)


def _combine_mm3_body(p_ref, dp_ref, r_ref, b_ref, w3_ref, b3_ref,
                      s_ref, t_ref):
    inv = _deg_inv(dp_ref)
    h = jnp.maximum((p_ref[0] + p_ref[1]) * inv + r_ref[...] + b_ref[...],
                    0.0)
    st = jnp.dot(h, w3_ref[...], preferred_element_type=jnp.float32)
    s_ref[...] = jnp.broadcast_to(st[:, 0:1], (BR, DD))
    t_ref[...] = jnp.broadcast_to(st[:, 1:2] + b3_ref[0, 0], (BR, W16))


def _final_body(q_ref, dp_ref, t_ref, o_ref):
    inv = _deg_inv(dp_ref)
    o_ref[...] = (q_ref[0, :, 0:W16] + q_ref[1, :, 0:W16]) * inv + t_ref[...]


def _row_spec(w):
    return pl.BlockSpec((BR, w), lambda i: (i, 0))


def _pair_spec(w):
    return pl.BlockSpec((2, BR, w), lambda i: (0, i, 0))


_DP_SPEC = pl.BlockSpec((BR, NW), lambda i: (i, 0))


def _full_spec(shape):
    nd = len(shape)
    return pl.BlockSpec(shape, lambda i, _n=nd: (0,) * _n)


def _mm2(x, wl, wr):
    return pl.pallas_call(
        _mm2_body,
        grid=(NB,),
        in_specs=[_row_spec(DD), _full_spec(wl.shape), _full_spec(wr.shape)],
        out_specs=[_row_spec(DD), _row_spec(DD)],
        out_shape=[jax.ShapeDtypeStruct((NP, DD), jnp.float32)] * 2,
    )(x, wl, wr)


def _combine_mm2(p, dp, r, b, wl, wr):
    b2 = b.reshape(1, DD)
    return pl.pallas_call(
        _combine_mm2_body,
        grid=(NB,),
        in_specs=[_pair_spec(DD), _DP_SPEC, _row_spec(DD),
                  _full_spec((1, DD)), _full_spec(wl.shape),
                  _full_spec(wr.shape)],
        out_specs=[_row_spec(DD), _row_spec(DD)],
        out_shape=[jax.ShapeDtypeStruct((NP, DD), jnp.float32)] * 2,
    )(p, dp, r, b2, wl, wr)


def _combine_mm3(p, dp, r, b, w3, b3):
    b2 = b.reshape(1, DD)
    b3r = b3.reshape(1, 1)
    return pl.pallas_call(
        _combine_mm3_body,
        grid=(NB,),
        in_specs=[_pair_spec(DD), _DP_SPEC, _row_spec(DD),
                  _full_spec((1, DD)), _full_spec(w3.shape),
                  _full_spec((1, 1))],
        out_specs=[_row_spec(DD), _row_spec(W16)],
        out_shape=[jax.ShapeDtypeStruct((NP, DD), jnp.float32),
                   jax.ShapeDtypeStruct((NP, W16), jnp.float32)],
    )(p, dp, r, b2, w3, b3r)


def _final(q, dp, t):
    return pl.pallas_call(
        _final_body,
        grid=(NB,),
        in_specs=[_pair_spec(DD), _DP_SPEC, _row_spec(W16)],
        out_specs=_row_spec(W16),
        out_shape=jax.ShapeDtypeStruct((NP, W16), jnp.float32),
    )(q, dp, t)


@jax.jit
def kernel(x, edge_index, Wl1, Wr1, b1, Wl2, Wr2, b2, Wl3, Wr3, b3):
    src = edge_index[0]
    dst = edge_index[1]
    x = jnp.pad(x, ((0, NP - NN), (0, 0)))

    seg_deg = _make_seg(DD, True)
    seg = _make_seg(DD, False)

    y1, r1 = _mm2(x, Wl1, Wr1)
    p1, dp4 = seg_deg(y1, src, dst)
    dp = jnp.transpose(dp4.reshape(NW, NP))
    y2, r2 = _combine_mm2(p1, dp, r1, b1, Wl2, Wr2)
    p2, = seg(y2, src, dst)
    w3 = jnp.zeros((DD, W16), jnp.float32)
    w3 = w3.at[:, 0].set(Wl3[:, 0]).at[:, 1].set(Wr3[:, 0])
    s16, t16 = _combine_mm3(p2, dp, r2, b2, w3, b3)
    q, = seg(s16, src, dst)
    out16 = _final(q, dp, t16)
    return out16[:NN, 0]
